# Initial kernel scaffold; baseline (speedup 1.0000x reference)
import jax
import jax.numpy as jnp
from jax.experimental import pallas as pl


def kernel(z_i, z_j, ts_rate_i, ts_rate_j):
    raise NotImplementedError("write your pallas kernel here")


# speedup vs baseline: 289.0164x; 289.0164x over previous
import functools
import math

import jax
import jax.numpy as jnp
from jax import lax
from jax.experimental import pallas as pl
from jax.experimental.pallas import tpu as pltpu
from jax.experimental.pallas import tpu_sc as plsc

_C = 300
_LOG_BIN_WIDTH = (math.log(300.0) - math.log(20.0)) / _C

_NC, _NS, _L = 2, 16, 16
_NW = _NC * _NS
_ROWS_PER_CHUNK = 128


def _prep_body(zi_ref, zj_ref, ri_ref, rj_ref, lp_ref, p_ref, s_ref):
    zi = zi_ref[...]
    mi = jnp.max(zi, axis=-1, keepdims=True)
    ei = jnp.exp(zi - mi)
    lp_ref[...] = zi - (mi + jnp.log(jnp.sum(ei, axis=-1, keepdims=True)))
    zj = zj_ref[...]
    mj = jnp.max(zj, axis=-1, keepdims=True)
    ej = jnp.exp(zj - mj)
    p_ref[...] = ej / jnp.sum(ej, axis=-1, keepdims=True)
    shift_logs = jnp.log(rj_ref[...] / ri_ref[...]) / _LOG_BIN_WIDTH
    s_ref[...] = -jnp.round(shift_logs).astype(jnp.int32)


def _prep(z_i, z_j, r_i, r_j):
    B, C = z_i.shape
    R = 256
    return pl.pallas_call(
        _prep_body,
        grid=(B // R,),
        in_specs=[
            pl.BlockSpec((R, C), lambda i: (i, 0)),
            pl.BlockSpec((R, C), lambda i: (i, 0)),
            pl.BlockSpec((R, 1), lambda i: (i, 0)),
            pl.BlockSpec((R, 1), lambda i: (i, 0)),
        ],
        out_specs=[
            pl.BlockSpec((R, C), lambda i: (i, 0)),
            pl.BlockSpec((R, C), lambda i: (i, 0)),
            pl.BlockSpec((R, 1), lambda i: (i, 0)),
        ],
        out_shape=[
            jax.ShapeDtypeStruct((B, C), jnp.float32),
            jax.ShapeDtypeStruct((B, C), jnp.float32),
            jax.ShapeDtypeStruct((B, 1), jnp.int32),
        ],
    )(z_i, z_j, r_i, r_j)


def _sc_body(lp_hbm, p_hbm, s_hbm, out_hbm, lp_v, p_v, s_v, acc_v):
    wid = lax.axis_index("c") * _NS + lax.axis_index("s")
    rows_per_w = lp_hbm.shape[0] // _NW
    n_chunks = rows_per_w // _ROWS_PER_CHUNK
    groups = _ROWS_PER_CHUNK // _L
    lanes = lax.iota(jnp.int32, _L)

    def chunk_body(ch, total):
        base = wid * rows_per_w + ch * _ROWS_PER_CHUNK
        pltpu.sync_copy(lp_hbm.at[pl.ds(base, _ROWS_PER_CHUNK), :], lp_v)
        pltpu.sync_copy(p_hbm.at[pl.ds(base, _ROWS_PER_CHUNK), :], p_v)
        pltpu.sync_copy(s_hbm.at[pl.ds(base, _ROWS_PER_CHUNK)], s_v)

        def group_body(g, tot):
            rows = g * _L + lanes
            svec = s_v[pl.ds(g * _L, _L)]

            def m_body(m, acc):
                col = svec + m
                valid = (col >= 0) & (col < _C)
                colc = jnp.minimum(jnp.maximum(col, 0), _C - 1)
                vlp = plsc.load_gather(lp_v, [rows, colc])
                vp = plsc.load_gather(p_v, [rows, jnp.full((_L,), m, jnp.int32)])
                return acc + jnp.where(valid, vp * vlp, 0.0)

            acc = lax.fori_loop(0, _C, m_body, jnp.zeros((_L,), jnp.float32))
            return tot + acc

        return lax.fori_loop(0, groups, group_body, total)

    total = lax.fori_loop(0, n_chunks, chunk_body, jnp.zeros((_L,), jnp.float32))
    acc_v[...] = total
    pltpu.sync_copy(acc_v, out_hbm.at[wid])


def _sc_shifted_dot(lp, p, s):
    mesh = plsc.VectorSubcoreMesh(core_axis_name="c", subcore_axis_name="s")
    fn = functools.partial(
        pl.kernel,
        out_type=jax.ShapeDtypeStruct((_NW, _L), jnp.float32),
        mesh=mesh,
        scratch_types=[
            pltpu.VMEM((_ROWS_PER_CHUNK, _C), jnp.float32),
            pltpu.VMEM((_ROWS_PER_CHUNK, _C), jnp.float32),
            pltpu.VMEM((_ROWS_PER_CHUNK,), jnp.int32),
            pltpu.VMEM((_L,), jnp.float32),
        ],
        compiler_params=pltpu.CompilerParams(needs_layout_passes=False),
    )(_sc_body)
    return fn(lp, p, s)


def kernel(z_i, z_j, ts_rate_i, ts_rate_j):
    B = z_i.shape[0]
    lp, p, s2 = _prep(z_i, z_j, ts_rate_i, ts_rate_j)
    parts = _sc_shifted_dot(lp, p, s2.reshape(-1))
    return -jnp.sum(parts) / B


# speedup vs baseline: 319.7315x; 1.1063x over previous
import functools
import math

import jax
import jax.numpy as jnp
from jax import lax
from jax.experimental import pallas as pl
from jax.experimental.pallas import tpu as pltpu
from jax.experimental.pallas import tpu_sc as plsc

_C = 300
_LOG_BIN_WIDTH = (math.log(300.0) - math.log(20.0)) / _C

_NC, _NS, _L = 2, 16, 16
_NW = _NC * _NS
_PAD = 160
_CP = 304
_WP = 624
_CH = 32
_NBUF = 2


def _prep_body(zi_ref, zj_ref, ri_ref, rj_ref, lp_ref, p_ref, s_ref):
    zi = zi_ref[...]
    mi = jnp.max(zi, axis=-1, keepdims=True)
    ei = jnp.exp(zi - mi)
    lp = zi - (mi + jnp.log(jnp.sum(ei, axis=-1, keepdims=True)))
    lp_ref[...] = jnp.concatenate(
        [lp, jnp.zeros((lp.shape[0], _CP - _C), jnp.float32)], axis=1)
    zj = zj_ref[...]
    mj = jnp.max(zj, axis=-1, keepdims=True)
    ej = jnp.exp(zj - mj)
    p_ref[...] = ej / jnp.sum(ej, axis=-1, keepdims=True)
    shift_logs = jnp.log(rj_ref[...] / ri_ref[...]) / _LOG_BIN_WIDTH
    s_ref[...] = -jnp.round(shift_logs).astype(jnp.int32)


def _prep(z_i, z_j, r_i, r_j):
    B, C = z_i.shape
    R = 256
    return pl.pallas_call(
        _prep_body,
        grid=(B // R,),
        in_specs=[
            pl.BlockSpec((R, C), lambda i: (i, 0)),
            pl.BlockSpec((R, C), lambda i: (i, 0)),
            pl.BlockSpec((R, 1), lambda i: (i, 0)),
            pl.BlockSpec((R, 1), lambda i: (i, 0)),
        ],
        out_specs=[
            pl.BlockSpec((R, _CP), lambda i: (i, 0)),
            pl.BlockSpec((R, C), lambda i: (i, 0)),
            pl.BlockSpec((R, 1), lambda i: (i, 0)),
        ],
        out_shape=[
            jax.ShapeDtypeStruct((B, _CP), jnp.float32),
            jax.ShapeDtypeStruct((B, C), jnp.float32),
            jax.ShapeDtypeStruct((B, 1), jnp.int32),
        ],
    )(z_i, z_j, r_i, r_j)


def _sc_body(lp_hbm, p_hbm, s_hbm, out_hbm,
             lp0, lp1, p0, p1, s0, s1, acc_v, sem0, sem1):
    wid = lax.axis_index("c") * _NS + lax.axis_index("s")
    rows_per_w = lp_hbm.shape[0] // _NW
    n_chunks = rows_per_w // _CH
    groups = _CH // _L
    lanes = lax.iota(jnp.int32, _L)
    row0 = wid * rows_per_w
    lp_bufs, p_bufs, s_bufs, sems = (lp0, lp1), (p0, p1), (s0, s1), (sem0, sem1)
    zf = jnp.zeros((_L,), jnp.float32)

    def zero_row(r, _):
        rv = jnp.full((_L,), r, jnp.int32)
        for b in range(_NBUF):
            for j in range(10):
                plsc.store_scatter(lp_bufs[b], [rv, j * 16 + lanes], zf)
            for j in range(10):
                plsc.store_scatter(lp_bufs[b], [rv, 464 + j * 16 + lanes], zf)
        return 0
    lax.fori_loop(0, _CH, zero_row, 0)

    def start_chunk(ch, b):
        base = row0 + ch * _CH
        pltpu.make_async_copy(
            lp_hbm.at[pl.ds(base, _CH), :],
            lp_bufs[b].at[:, pl.ds(_PAD, _CP)], sems[b]).start()
        pltpu.make_async_copy(
            p_hbm.at[pl.ds(base, _CH), :], p_bufs[b], sems[b]).start()
        pltpu.make_async_copy(
            s_hbm.at[pl.ds(base, _CH)], s_bufs[b], sems[b]).start()

    def wait_chunk(ch, b):
        base = row0 + ch * _CH
        pltpu.make_async_copy(
            lp_hbm.at[pl.ds(base, _CH), :],
            lp_bufs[b].at[:, pl.ds(_PAD, _CP)], sems[b]).wait()
        pltpu.make_async_copy(
            p_hbm.at[pl.ds(base, _CH), :], p_bufs[b], sems[b]).wait()
        pltpu.make_async_copy(
            s_hbm.at[pl.ds(base, _CH)], s_bufs[b], sems[b]).wait()

    start_chunk(0, 0)

    def outer(k, total):
        for b in range(_NBUF):
            ch = _NBUF * k + b
            @pl.when(ch + 1 < n_chunks)
            def _():
                start_chunk(ch + 1, (b + 1) % _NBUF)
            wait_chunk(ch, b)
            for g in range(groups):
                rows = g * _L + lanes
                svec = s_bufs[b][pl.ds(g * _L, _L)]
                colbase = svec + _PAD

                def m_body(m, acc):
                    mv = jnp.full((_L,), m, jnp.int32)
                    vlp = plsc.load_gather(lp_bufs[b], [rows, colbase + mv])
                    vp = plsc.load_gather(p_bufs[b], [rows, mv])
                    return acc + vp * vlp

                total = total + lax.fori_loop(
                    0, _C, m_body, jnp.zeros((_L,), jnp.float32), unroll=6)
        return total

    total = lax.fori_loop(0, n_chunks // _NBUF, outer,
                          jnp.zeros((_L,), jnp.float32))
    acc_v[...] = total
    pltpu.sync_copy(acc_v, out_hbm.at[wid])


def _sc_shifted_dot(lp, p, s):
    mesh = plsc.VectorSubcoreMesh(core_axis_name="c", subcore_axis_name="s")
    fn = functools.partial(
        pl.kernel,
        out_type=jax.ShapeDtypeStruct((_NW, _L), jnp.float32),
        mesh=mesh,
        scratch_types=[
            pltpu.VMEM((_CH, _WP), jnp.float32),
            pltpu.VMEM((_CH, _WP), jnp.float32),
            pltpu.VMEM((_CH, _C), jnp.float32),
            pltpu.VMEM((_CH, _C), jnp.float32),
            pltpu.VMEM((_CH,), jnp.int32),
            pltpu.VMEM((_CH,), jnp.int32),
            pltpu.VMEM((_L,), jnp.float32),
            pltpu.SemaphoreType.DMA,
            pltpu.SemaphoreType.DMA,
        ],
        compiler_params=pltpu.CompilerParams(
            needs_layout_passes=False, use_tc_tiling_on_sc=False),
    )(_sc_body)
    return fn(lp, p, s)


def kernel(z_i, z_j, ts_rate_i, ts_rate_j):
    B = z_i.shape[0]
    lp, p, s2 = _prep(z_i, z_j, ts_rate_i, ts_rate_j)
    parts = _sc_shifted_dot(lp, p, s2.reshape(-1))
    return -jnp.sum(parts) / B
